# baseline (speedup 1.0000x reference)
import math

import jax
import jax.numpy as jnp
from jax.experimental import pallas as pl

_D = 128
_DEPTH = 3
_RATIO = 0.5
_BM = 256
_BK = 256


def _pad_to(n, m):
    return ((n + m - 1) // m) * m


def _gcn_mm_body(a_ref, u_ref, acc_ref):
    i = pl.program_id(0)
    k = pl.program_id(1)

    @pl.when(k == 0)
    def _init():
        acc_ref[...] = jnp.zeros_like(acc_ref)

    acc_ref[...] += jax.lax.dot_general(
        a_ref[...], u_ref[...], (((1,), (0,)), ((), ())),
        preferred_element_type=jnp.float32,
        precision=jax.lax.Precision.HIGHEST)

    @pl.when(k == i)
    def _diag():
        acc_ref[...] += 2.0 * u_ref[...]


def _gcn_matmul(A, U):
    n = A.shape[0]
    return pl.pallas_call(
        _gcn_mm_body,
        grid=(n // _BM, n // _BK),
        in_specs=[
            pl.BlockSpec((_BM, _BK), lambda i, k: (i, k)),
            pl.BlockSpec((_BK, _D), lambda i, k: (k, 0)),
        ],
        out_specs=pl.BlockSpec((_BM, _D), lambda i, k: (i, 0)),
        out_shape=jax.ShapeDtypeStruct((n, _D), jnp.float32),
    )(A, U)


def _adj_mm_body(rows_ref, cols_ref, out_ref):
    i = pl.program_id(0)
    k = pl.program_id(1)
    nk = pl.num_programs(1)

    @pl.when(k == 0)
    def _init():
        out_ref[...] = jnp.zeros_like(out_ref)

    out_ref[...] += jax.lax.dot_general(
        rows_ref[...], cols_ref[...], (((1,), (0,)), ((), ())),
        preferred_element_type=jnp.float32)

    @pl.when(k == nk - 1)
    def _zero_diag():
        r = jax.lax.broadcasted_iota(jnp.int32, out_ref.shape, 0) + i * _BM
        c = jax.lax.broadcasted_iota(jnp.int32, out_ref.shape, 1)
        out_ref[...] = jnp.where(r == c, 0.0, out_ref[...])


def _adj_matmul(rows, cols):
    m, K = rows.shape
    return pl.pallas_call(
        _adj_mm_body,
        grid=(m // _BM, K // _BK),
        in_specs=[
            pl.BlockSpec((_BM, _BK), lambda i, k: (i, k)),
            pl.BlockSpec((_BK, m), lambda i, k: (k, 0)),
        ],
        out_specs=pl.BlockSpec((_BM, m), lambda i, k: (i, 0)),
        out_shape=jax.ShapeDtypeStruct((m, m), jnp.float32),
    )(rows, cols)


def _xw_body(x_ref, w_ref, o_ref):
    o_ref[...] = jax.lax.dot_general(
        x_ref[...], w_ref[...], (((1,), (0,)), ((), ())),
        preferred_element_type=jnp.float32,
        precision=jax.lax.Precision.HIGHEST)


def _xw_matmul(x, W):
    n = x.shape[0]
    return pl.pallas_call(
        _xw_body,
        grid=(n // _BM,),
        in_specs=[
            pl.BlockSpec((_BM, _D), lambda i: (i, 0)),
            pl.BlockSpec((_D, _D), lambda i: (0, 0)),
        ],
        out_specs=pl.BlockSpec((_BM, _D), lambda i: (i, 0)),
        out_shape=jax.ShapeDtypeStruct((n, _D), jnp.float32),
    )(x, W)


def _gcn_layer(A, dinv, valid, h, W, b, relu):
    z = _xw_matmul(h, W)
    u = dinv[:, None] * z
    acc = _gcn_matmul(A, u)
    out = dinv[:, None] * acc + b[None, :]
    out = jnp.where(valid[:, None], out, 0.0)
    if relu:
        out = jnp.maximum(out, 0.0)
    return out


def _degrees(A, n_real):
    npad = A.shape[0]
    valid = jnp.arange(npad) < n_real
    deg = A.sum(axis=1) + 2.0 * valid
    dinv = jnp.where(deg > 0.0, 1.0 / jnp.sqrt(deg), 0.0)
    return dinv, valid


def kernel(x, edge_index, Wd0, bd0, Wd1, bd1, Wd2, bd2, Wd3, bd3,
           pw0, pw1, pw2, Wu0, bu0, Wu1, bu1, Wu2, bu2):
    Wd = [Wd0, Wd1, Wd2, Wd3]
    bd = [bd0, bd1, bd2, bd3]
    pw = [pw0, pw1, pw2]
    Wu = [Wu0, Wu1, Wu2]
    bu = [bu0, bu1, bu2]

    n0 = x.shape[0]
    n0p = _pad_to(n0, _BM)

    src = edge_index[0]
    dst = edge_index[1]
    A = jnp.zeros((n0p, n0p), jnp.float32).at[src, dst].add(1.0)
    deg0 = jnp.zeros((n0p,), jnp.float32).at[src].add(1.0)
    valid0 = jnp.arange(n0p) < n0
    deg0 = deg0 + 2.0 * valid0
    dinv = jnp.where(deg0 > 0.0, 1.0 / jnp.sqrt(deg0), 0.0)
    valid = valid0

    hp = jnp.zeros((n0p, _D), x.dtype).at[:n0, :].set(x)
    h = _gcn_layer(A, dinv, valid, hp, Wd[0], bd[0], relu=True)

    n_real = n0
    xs = [h]
    As = [A]
    dinvs = [dinv]
    n_reals = [n0]
    perms = []

    for i in range(1, _DEPTH + 1):
        npad = A.shape[0]
        g = (h @ pw[i - 1]) / jnp.linalg.norm(pw[i - 1])
        score = jnp.tanh(g)
        score = jnp.where(jnp.arange(npad) < n_real, score, -jnp.inf)
        k = int(math.ceil(_RATIO * n_real))
        _, perm = jax.lax.top_k(score, k)
        kp = _pad_to(k, _BM)
        perm_pad = jnp.concatenate(
            [perm, jnp.full((kp - k,), n_real, jnp.int32)])
        kvalid = jnp.arange(kp) < k

        Brows = A[perm_pad]
        Brows = Brows.at[jnp.arange(k), perm].add(1.0)
        Bcols = A[:, perm_pad]
        Bcols = Bcols.at[perm, jnp.arange(k)].add(1.0)
        A = _adj_matmul(Brows.astype(jnp.bfloat16), Bcols.astype(jnp.bfloat16))

        sg = jnp.where(kvalid, score[perm_pad], 0.0)
        h = h[perm_pad] * sg[:, None]

        dinv, valid = _degrees(A, k)
        h = _gcn_layer(A, dinv, valid, h, Wd[i], bd[i], relu=True)
        n_real = k
        perms.append(perm)
        if i < _DEPTH:
            xs.append(h)
            As.append(A)
            dinvs.append(dinv)
            n_reals.append(k)

    ks = [p.shape[0] for p in perms]
    for i in range(_DEPTH):
        j = _DEPTH - 1 - i
        res = xs[j]
        up = jnp.zeros_like(res).at[perms[j]].set(h[:ks[j]])
        h = res + up
        valid_j = jnp.arange(As[j].shape[0]) < n_reals[j]
        h = _gcn_layer(As[j], dinvs[j], valid_j, h, Wu[i], bu[i],
                       relu=(i < _DEPTH - 1))

    return h[:n0]
